# baseline (speedup 1.0000x reference)
import functools

import numpy as np
import jax
import jax.numpy as jnp
from jax import lax
from jax.experimental import pallas as pl
from jax.experimental.pallas import tpu as pltpu
from jax.experimental.pallas import tpu_sc as plsc

F = 128
NC, NS = 2, 16
NW = NC * NS
NL = 320
NPAD = NW * NL
C = 1024
G = 128
NB = C // G
CHK_A = 20
EPAD = 16 * CHK_A * C
CHK_B = EPAD // C
BIG = 3.0e38

_DEG_HIST = np.array([0.0, 1000.0, 2000.0, 4000.0, 2000.0, 1000.0])
_BINS = np.arange(_DEG_HIST.shape[0], dtype=np.float64)
AVG_DEG_LOG = float((np.log(_BINS + 1.0) * _DEG_HIST).sum() / _DEG_HIST.sum())

_P = jax.lax.Precision.HIGHEST


def _dot(a, b):
    return jnp.dot(a, b, preferred_element_type=jnp.float32, precision=_P)


def _pre_body(x_ref, w1_ref, w2_ref, b_ref, a_ref, bout_ref, b2_ref):
    xb = x_ref[...]
    a_ref[...] = _dot(xb, w1_ref[...]) + b_ref[...]
    bv = _dot(xb, w2_ref[...])
    bout_ref[...] = bv
    b2_ref[...] = bv * bv


def _pre(x_pad, W1, W2, b):
    R = 1024
    return pl.pallas_call(
        _pre_body,
        grid=(NPAD // R,),
        in_specs=[
            pl.BlockSpec((R, F), lambda i: (i, 0)),
            pl.BlockSpec((F, F), lambda i: (0, 0)),
            pl.BlockSpec((F, F), lambda i: (0, 0)),
            pl.BlockSpec((1, F), lambda i: (0, 0)),
        ],
        out_specs=[
            pl.BlockSpec((R, F), lambda i: (i, 0)),
            pl.BlockSpec((R, F), lambda i: (i, 0)),
            pl.BlockSpec((R, F), lambda i: (i, 0)),
        ],
        out_shape=[
            jax.ShapeDtypeStruct((NPAD, F), jnp.float32),
            jax.ShapeDtypeStruct((NPAD, F), jnp.float32),
            jax.ShapeDtypeStruct((NPAD, F), jnp.float32),
        ],
    )(x_pad, W1, W2, b)


def _mesh():
    return plsc.VectorSubcoreMesh(core_axis_name="c", subcore_axis_name="s",
                                  num_cores=NC, num_subcores=NS)


def _sc_sums(B_tab, B2_tab, src_1d, dst_2d):
    @functools.partial(
        pl.kernel,
        out_type=[
            jax.ShapeDtypeStruct((NPAD, F), jnp.float32),
            jax.ShapeDtypeStruct((NPAD, F), jnp.float32),
        ],
        mesh=_mesh(),
        scratch_types=[
            pltpu.VMEM((NB, G), jnp.int32),
            pltpu.VMEM((C,), jnp.int32),
            pltpu.VMEM((G, F), jnp.float32),
            pltpu.VMEM_SHARED((NPAD, F), jnp.float32),
            pltpu.SemaphoreType.DMA,
        ],
    )
    def body(tab, tab2, srch, dsth2, o_sum, o_sq,
             st_d2, st_s, grows, shacc, sem):
        cid = lax.axis_index("c")
        sid = lax.axis_index("s")
        zeros16 = jnp.zeros((16,), jnp.float32)

        def zg(i, _):
            for g in range(F // 16):
                grows[i, pl.ds(g * 16, 16)] = zeros16
            return 0
        lax.fori_loop(0, G, zg, 0)

        rows_per_sub = NPAD // NS
        def zs(i, _):
            pltpu.sync_copy(
                grows,
                shacc.at[pl.ds(pl.multiple_of(sid * rows_per_sub + i * G, G),
                               G)])
            return 0
        lax.fori_loop(0, rows_per_sub // G, zs, 0)
        plsc.subcore_barrier()

        def phase_a(table):
            ebase = sid * (CHK_A * C)

            def chunk(k, _):
                coff = ebase + k * C
                pltpu.sync_copy(
                    dsth2.at[pl.ds(pl.multiple_of(coff // G, NB), NB)],
                    st_d2)
                pltpu.sync_copy(srch.at[pl.ds(coff, C)], st_s)

                def batch(b, _):
                    pltpu.async_copy(
                        table.at[st_s.at[pl.ds(b * G, G)]], grows, sem).wait()
                    pltpu.sync_copy(grows, shacc.at[st_d2.at[b]], add=True)
                    return 0
                lax.fori_loop(0, NB, batch, 0)
                return 0
            lax.fori_loop(0, CHK_A, chunk, 0)

        @pl.when(cid == 0)
        def _():
            phase_a(tab)

        @pl.when(cid == 1)
        def _():
            phase_a(tab2)

        plsc.subcore_barrier()

        obase = pl.multiple_of(sid * rows_per_sub, rows_per_sub)

        @pl.when(cid == 0)
        def _():
            pltpu.sync_copy(shacc.at[pl.ds(obase, rows_per_sub)],
                            o_sum.at[pl.ds(obase, rows_per_sub)])

        @pl.when(cid == 1)
        def _():
            pltpu.sync_copy(shacc.at[pl.ds(obase, rows_per_sub)],
                            o_sq.at[pl.ds(obase, rows_per_sub)])

    return body(B_tab, B2_tab, src_1d, dst_2d)


def _sc_minmax(B_tab, src_1d, dst_2d):
    @functools.partial(
        pl.kernel,
        out_type=[
            jax.ShapeDtypeStruct((NPAD, F), jnp.float32),
            jax.ShapeDtypeStruct((NPAD, F), jnp.float32),
            jax.ShapeDtypeStruct((NPAD * 16,), jnp.float32),
        ],
        mesh=_mesh(),
        scratch_types=[
            pltpu.VMEM((NL, F), jnp.float32),
            pltpu.VMEM((NL, F), jnp.float32),
            pltpu.VMEM((NL * 16,), jnp.float32),
            pltpu.VMEM((1, F), jnp.float32),
            pltpu.VMEM((NB, G), jnp.int32),
            pltpu.VMEM((C,), jnp.int32),
            pltpu.VMEM((16,), jnp.int32),
            pltpu.SemaphoreType.DMA,
        ],
    )
    def body(tab, srch, dsth2, o_mn, o_mx, o_cnt,
             a_mn, a_mx, a_cnt, rowb, st_d2, st_s, tmpm, sem):
        cid = lax.axis_index("c")
        sid = lax.axis_index("s")
        wid = cid * NS + sid
        zeros16 = jnp.zeros((16,), jnp.float32)
        ones16 = jnp.ones((16,), jnp.float32)
        big16 = jnp.full((16,), BIG, jnp.float32)

        nbase = wid * NL
        nbase_a = pl.multiple_of(wid * NL, NL)

        def init_acc(i, _):
            for g in range(F // 16):
                a_mn[i, pl.ds(g * 16, 16)] = big16
                a_mx[i, pl.ds(g * 16, 16)] = -big16
            a_cnt[pl.ds(i * 16, 16)] = zeros16
            return 0
        lax.fori_loop(0, NL, init_acc, 0)

        def scan_chunk(k, _):
            pltpu.sync_copy(
                dsth2.at[pl.ds(pl.multiple_of(k * NB, NB), NB)], st_d2)
            pltpu.sync_copy(srch.at[pl.ds(k * C, C)], st_s)

            def brow(b, _):
                for g in range(G // 16):
                    d16 = st_d2[b, pl.ds(g * 16, 16)]
                    dl16 = d16 - nbase
                    msk = (dl16 >= 0) & (dl16 < NL)
                    tmpm[pl.ds(0, 16)] = jnp.where(msk, 1, 0)
                    mi = tmpm[pl.ds(0, 16)]
                    s16 = st_s[pl.ds(b * G + g * 16, 16)]
                    for j in range(16):

                        @pl.when(mi[j] > 0)
                        def _():
                            dl = d16[j] - nbase
                            pltpu.sync_copy(tab.at[s16[j]], rowb.at[0])
                            for q in range(F // 16):
                                v = rowb[0, pl.ds(q * 16, 16)]
                                mn = a_mn[dl, pl.ds(q * 16, 16)]
                                a_mn[dl, pl.ds(q * 16, 16)] = (
                                    jnp.minimum(mn, v))
                                mx = a_mx[dl, pl.ds(q * 16, 16)]
                                a_mx[dl, pl.ds(q * 16, 16)] = (
                                    jnp.maximum(mx, v))
                            a_cnt[pl.ds(dl * 16, 16)] = (
                                a_cnt[pl.ds(dl * 16, 16)] + ones16)
                return 0
            lax.fori_loop(0, NB, brow, 0)
            return 0
        lax.fori_loop(0, CHK_B, scan_chunk, 0)

        pltpu.sync_copy(a_mn, o_mn.at[pl.ds(nbase_a, NL)])
        pltpu.sync_copy(a_mx, o_mx.at[pl.ds(nbase_a, NL)])
        pltpu.sync_copy(
            a_cnt,
            o_cnt.at[pl.ds(pl.multiple_of(wid * NL * 16, NL * 16),
                           NL * 16)])

    return body(B_tab, src_1d, dst_2d)


def _post_body(x_ref, a_ref, s1_ref, s2_ref, mn_ref, mx_ref, cnt_ref,
               wx_ref, wa_ref, wamp_ref, watt_ref, bp_ref, wl_ref, bl_ref,
               o_ref):
    cnt = cnt_ref[...][:, :1]
    A = a_ref[...]
    S1 = s1_ref[...]
    S2 = s2_ref[...]
    cc = jnp.maximum(cnt, 1.0)
    empty = cnt == 0.0
    s_sum = cnt * A + S1
    s_min = jnp.where(empty, 0.0, A + mn_ref[...])
    s_max = jnp.where(empty, 0.0, A + mx_ref[...])
    m1 = S1 / cc
    var = S2 / cc - m1 * m1
    s_std = jnp.sqrt(jnp.maximum(var, 0.0) + 1e-5)
    logd = jnp.log(cc + 1.0)
    amp = logd / AVG_DEG_LOG
    att = AVG_DEG_LOG / logd
    agg = jnp.concatenate([s_sum, s_min, s_max, s_std], axis=-1)
    t = (_dot(x_ref[...], wx_ref[...])
         + _dot(agg, wa_ref[...])
         + amp * _dot(agg, wamp_ref[...])
         + att * _dot(agg, watt_ref[...])
         + bp_ref[...])
    o_ref[...] = _dot(t, wl_ref[...]) + bl_ref[...]


def _post(x, A, S1, S2, Mn, Mx, CNT, Wx, Wa, Wamp, Watt, bp, Wl, bl):
    n = x.shape[0]
    R = 1000
    full = lambda r, c: pl.BlockSpec((r, c), lambda i: (0, 0))
    blk = lambda c: pl.BlockSpec((R, c), lambda i: (i, 0))
    return pl.pallas_call(
        _post_body,
        grid=(n // R,),
        in_specs=[
            blk(F), blk(F), blk(F), blk(F), blk(F), blk(F), blk(16),
            full(F, F), full(4 * F, F), full(4 * F, F), full(4 * F, F),
            full(1, F), full(F, F), full(1, F),
        ],
        out_specs=blk(F),
        out_shape=jax.ShapeDtypeStruct((n, F), jnp.float32),
    )(x, A, S1, S2, Mn, Mx, CNT, Wx, Wa, Wamp, Watt, bp, Wl, bl)


def kernel(x, edge_index, W_pre, b_pre, W_post, b_post, W_lin, b_lin):
    n, f = x.shape
    E = edge_index.shape[1]
    assert f == F and n <= NPAD and E <= EPAD
    x_pad = jnp.pad(x, ((0, NPAD - n), (0, 0)))
    A_pad, B_pad, B2_pad = _pre(x_pad, W_pre[:F], W_pre[F:],
                                b_pre.reshape(1, F))
    src = jnp.pad(edge_index[0], (0, EPAD - E))
    dst = jnp.pad(edge_index[1], (0, EPAD - E),
                  constant_values=NPAD - 1)
    dst2 = dst.reshape(EPAD // G, G)
    S1, S2 = _sc_sums(B_pad, B2_pad, src, dst2)
    Mn, Mx, CNT = _sc_minmax(B_pad, src, dst2)
    CNT2 = CNT.reshape(NPAD, 16)
    return _post(x, A_pad[:n], S1[:n], S2[:n], Mn[:n], Mx[:n], CNT2[:n],
                 W_post[:F], W_post[F:5 * F], W_post[5 * F:9 * F],
                 W_post[9 * F:13 * F], b_post.reshape(1, F),
                 W_lin, b_lin.reshape(1, F))


# speedup vs baseline: 3.7223x; 3.7223x over previous
import functools

import numpy as np
import jax
import jax.numpy as jnp
from jax import lax
from jax.experimental import pallas as pl
from jax.experimental.pallas import tpu as pltpu
from jax.experimental.pallas import tpu_sc as plsc

F = 128
NC, NS = 2, 16
NW = NC * NS
NL = 320
NPAD = NW * NL
C = 1024
G = 128
NB = C // G
CHK_A = 20
EPAD = 16 * CHK_A * C
CHK_B = EPAD // C
QD = 128
BIG = 3.0e38

_DEG_HIST = np.array([0.0, 1000.0, 2000.0, 4000.0, 2000.0, 1000.0])
_BINS = np.arange(_DEG_HIST.shape[0], dtype=np.float64)
AVG_DEG_LOG = float((np.log(_BINS + 1.0) * _DEG_HIST).sum() / _DEG_HIST.sum())

_P = jax.lax.Precision.HIGHEST


def _dot(a, b):
    return jnp.dot(a, b, preferred_element_type=jnp.float32, precision=_P)


def _pre_body(x_ref, w1_ref, w2_ref, b_ref, a_ref, bout_ref, b2_ref):
    xb = x_ref[...]
    a_ref[...] = _dot(xb, w1_ref[...]) + b_ref[...]
    bv = _dot(xb, w2_ref[...])
    bout_ref[...] = bv
    b2_ref[...] = bv * bv


def _pre(x_pad, W1, W2, b):
    R = 1024
    return pl.pallas_call(
        _pre_body,
        grid=(NPAD // R,),
        in_specs=[
            pl.BlockSpec((R, F), lambda i: (i, 0)),
            pl.BlockSpec((F, F), lambda i: (0, 0)),
            pl.BlockSpec((F, F), lambda i: (0, 0)),
            pl.BlockSpec((1, F), lambda i: (0, 0)),
        ],
        out_specs=[
            pl.BlockSpec((R, F), lambda i: (i, 0)),
            pl.BlockSpec((R, F), lambda i: (i, 0)),
            pl.BlockSpec((R, F), lambda i: (i, 0)),
        ],
        out_shape=[
            jax.ShapeDtypeStruct((NPAD, F), jnp.float32),
            jax.ShapeDtypeStruct((NPAD, F), jnp.float32),
            jax.ShapeDtypeStruct((NPAD, F), jnp.float32),
        ],
    )(x_pad, W1, W2, b)


def _mesh():
    return plsc.VectorSubcoreMesh(core_axis_name="c", subcore_axis_name="s",
                                  num_cores=NC, num_subcores=NS)


def _sc_sums(B_tab, B2_tab, src_1d, dst_2d):
    @functools.partial(
        pl.kernel,
        out_type=[
            jax.ShapeDtypeStruct((NPAD, F), jnp.float32),
            jax.ShapeDtypeStruct((NPAD, F), jnp.float32),
        ],
        mesh=_mesh(),
        scratch_types=[
            pltpu.VMEM((NB, G), jnp.int32),
            pltpu.VMEM((C,), jnp.int32),
            pltpu.VMEM((G, F), jnp.float32),
            pltpu.VMEM_SHARED((NPAD, F), jnp.float32),
            pltpu.SemaphoreType.DMA,
        ],
    )
    def body(tab, tab2, srch, dsth2, o_sum, o_sq,
             st_d2, st_s, grows, shacc, sem):
        cid = lax.axis_index("c")
        sid = lax.axis_index("s")
        zeros16 = jnp.zeros((16,), jnp.float32)

        def zg(i, _):
            for g in range(F // 16):
                grows[i, pl.ds(g * 16, 16)] = zeros16
            return 0
        lax.fori_loop(0, G, zg, 0)

        rows_per_sub = NPAD // NS
        def zs(i, _):
            pltpu.sync_copy(
                grows,
                shacc.at[pl.ds(pl.multiple_of(sid * rows_per_sub + i * G, G),
                               G)])
            return 0
        lax.fori_loop(0, rows_per_sub // G, zs, 0)
        plsc.subcore_barrier()

        def phase_a(table):
            ebase = sid * (CHK_A * C)

            def chunk(k, _):
                coff = ebase + k * C
                pltpu.sync_copy(
                    dsth2.at[pl.ds(pl.multiple_of(coff // G, NB), NB)],
                    st_d2)
                pltpu.sync_copy(srch.at[pl.ds(coff, C)], st_s)

                def batch(b, _):
                    pltpu.async_copy(
                        table.at[st_s.at[pl.ds(b * G, G)]], grows, sem).wait()
                    pltpu.sync_copy(grows, shacc.at[st_d2.at[b]], add=True)
                    return 0
                lax.fori_loop(0, NB, batch, 0)
                return 0
            lax.fori_loop(0, CHK_A, chunk, 0)

        @pl.when(cid == 0)
        def _():
            phase_a(tab)

        @pl.when(cid == 1)
        def _():
            phase_a(tab2)

        plsc.subcore_barrier()

        obase = pl.multiple_of(sid * rows_per_sub, rows_per_sub)

        @pl.when(cid == 0)
        def _():
            pltpu.sync_copy(shacc.at[pl.ds(obase, rows_per_sub)],
                            o_sum.at[pl.ds(obase, rows_per_sub)])

        @pl.when(cid == 1)
        def _():
            pltpu.sync_copy(shacc.at[pl.ds(obase, rows_per_sub)],
                            o_sq.at[pl.ds(obase, rows_per_sub)])

    return body(B_tab, B2_tab, src_1d, dst_2d)


def _sc_minmax(B_tab, src_1d, dst_2d):
    @functools.partial(
        pl.kernel,
        out_type=[
            jax.ShapeDtypeStruct((NPAD, F), jnp.float32),
            jax.ShapeDtypeStruct((NPAD, F), jnp.float32),
            jax.ShapeDtypeStruct((NPAD * 16,), jnp.float32),
        ],
        mesh=_mesh(),
        scratch_types=[
            pltpu.VMEM((NL, F), jnp.float32),
            pltpu.VMEM((NL, F), jnp.float32),
            pltpu.VMEM((NL * 16,), jnp.float32),
            pltpu.VMEM((QD, F), jnp.float32),
            pltpu.VMEM((NB, G), jnp.int32),
            pltpu.VMEM((C,), jnp.int32),
            pltpu.VMEM((16,), jnp.int32),
            pltpu.SMEM((QD + 1,), jnp.int32),
            pltpu.SemaphoreType.DMA,
        ],
    )
    def body(tab, srch, dsth2, o_mn, o_mx, o_cnt,
             a_mn, a_mx, a_cnt, rowq, st_d2, st_s, tmpm, smq, sem):
        cid = lax.axis_index("c")
        sid = lax.axis_index("s")
        wid = cid * NS + sid
        zeros16 = jnp.zeros((16,), jnp.float32)
        ones16 = jnp.ones((16,), jnp.float32)
        big16 = jnp.full((16,), BIG, jnp.float32)

        nbase = wid * NL
        nbase_a = pl.multiple_of(wid * NL, NL)

        def init_acc(i, _):
            for g in range(F // 16):
                a_mn[i, pl.ds(g * 16, 16)] = big16
                a_mx[i, pl.ds(g * 16, 16)] = -big16
            a_cnt[pl.ds(i * 16, 16)] = zeros16
            return 0
        lax.fori_loop(0, NL, init_acc, 0)
        smq[QD] = 0

        def flush():
            qc = smq[QD]

            def drain(i, _):
                @pl.when(i < qc)
                def _():
                    pltpu.make_async_copy(tab.at[0], rowq.at[i], sem).wait()
                return 0
            lax.fori_loop(0, QD, drain, 0)

            def proc(i, _):
                @pl.when(i < qc)
                def _():
                    dl = smq[i]
                    for q in range(F // 16):
                        v = rowq[i, pl.ds(q * 16, 16)]
                        mn = a_mn[dl, pl.ds(q * 16, 16)]
                        a_mn[dl, pl.ds(q * 16, 16)] = jnp.minimum(mn, v)
                        mx = a_mx[dl, pl.ds(q * 16, 16)]
                        a_mx[dl, pl.ds(q * 16, 16)] = jnp.maximum(mx, v)
                return 0
            lax.fori_loop(0, QD, proc, 0)
            smq[QD] = 0

        def scan_chunk(k, _):
            pltpu.sync_copy(
                dsth2.at[pl.ds(pl.multiple_of(k * NB, NB), NB)], st_d2)
            pltpu.sync_copy(srch.at[pl.ds(k * C, C)], st_s)

            def brow(b, _):
                for g in range(G // 16):
                    d16 = st_d2[b, pl.ds(g * 16, 16)]
                    dl16 = d16 - nbase
                    msk = (dl16 >= 0) & (dl16 < NL)
                    tmpm[pl.ds(0, 16)] = jnp.where(msk, 1, 0)
                    mi = tmpm[pl.ds(0, 16)]
                    s16 = st_s[pl.ds(b * G + g * 16, 16)]
                    for j in range(16):

                        @pl.when(mi[j] > 0)
                        def _():
                            dl = d16[j] - nbase
                            qp = smq[QD]
                            pltpu.async_copy(tab.at[s16[j]], rowq.at[qp],
                                             sem)
                            smq[qp] = dl
                            smq[QD] = qp + 1
                            a_cnt[pl.ds(dl * 16, 16)] = (
                                a_cnt[pl.ds(dl * 16, 16)] + ones16)

                    @pl.when(smq[QD] >= QD - 16)
                    def _():
                        flush()
                return 0
            lax.fori_loop(0, NB, brow, 0)
            return 0
        lax.fori_loop(0, CHK_B, scan_chunk, 0)
        flush()

        pltpu.sync_copy(a_mn, o_mn.at[pl.ds(nbase_a, NL)])
        pltpu.sync_copy(a_mx, o_mx.at[pl.ds(nbase_a, NL)])
        pltpu.sync_copy(
            a_cnt,
            o_cnt.at[pl.ds(pl.multiple_of(wid * NL * 16, NL * 16),
                           NL * 16)])

    return body(B_tab, src_1d, dst_2d)


def _post_body(x_ref, a_ref, s1_ref, s2_ref, mn_ref, mx_ref, cnt_ref,
               wx_ref, wa_ref, wamp_ref, watt_ref, bp_ref, wl_ref, bl_ref,
               o_ref):
    cnt = cnt_ref[...][:, :1]
    A = a_ref[...]
    S1 = s1_ref[...]
    S2 = s2_ref[...]
    cc = jnp.maximum(cnt, 1.0)
    empty = cnt == 0.0
    s_sum = cnt * A + S1
    s_min = jnp.where(empty, 0.0, A + mn_ref[...])
    s_max = jnp.where(empty, 0.0, A + mx_ref[...])
    m1 = S1 / cc
    var = S2 / cc - m1 * m1
    s_std = jnp.sqrt(jnp.maximum(var, 0.0) + 1e-5)
    logd = jnp.log(cc + 1.0)
    amp = logd / AVG_DEG_LOG
    att = AVG_DEG_LOG / logd
    agg = jnp.concatenate([s_sum, s_min, s_max, s_std], axis=-1)
    t = (_dot(x_ref[...], wx_ref[...])
         + _dot(agg, wa_ref[...])
         + amp * _dot(agg, wamp_ref[...])
         + att * _dot(agg, watt_ref[...])
         + bp_ref[...])
    o_ref[...] = _dot(t, wl_ref[...]) + bl_ref[...]


def _post(x, A, S1, S2, Mn, Mx, CNT, Wx, Wa, Wamp, Watt, bp, Wl, bl):
    n = x.shape[0]
    R = 1000
    full = lambda r, c: pl.BlockSpec((r, c), lambda i: (0, 0))
    blk = lambda c: pl.BlockSpec((R, c), lambda i: (i, 0))
    return pl.pallas_call(
        _post_body,
        grid=(n // R,),
        in_specs=[
            blk(F), blk(F), blk(F), blk(F), blk(F), blk(F), blk(16),
            full(F, F), full(4 * F, F), full(4 * F, F), full(4 * F, F),
            full(1, F), full(F, F), full(1, F),
        ],
        out_specs=blk(F),
        out_shape=jax.ShapeDtypeStruct((n, F), jnp.float32),
    )(x, A, S1, S2, Mn, Mx, CNT, Wx, Wa, Wamp, Watt, bp, Wl, bl)


def kernel(x, edge_index, W_pre, b_pre, W_post, b_post, W_lin, b_lin):
    n, f = x.shape
    E = edge_index.shape[1]
    assert f == F and n <= NPAD and E <= EPAD
    x_pad = jnp.pad(x, ((0, NPAD - n), (0, 0)))
    A_pad, B_pad, B2_pad = _pre(x_pad, W_pre[:F], W_pre[F:],
                                b_pre.reshape(1, F))
    src = jnp.pad(edge_index[0], (0, EPAD - E))
    dst = jnp.pad(edge_index[1], (0, EPAD - E),
                  constant_values=NPAD - 1)
    dst2 = dst.reshape(EPAD // G, G)
    S1, S2 = _sc_sums(B_pad, B2_pad, src, dst2)
    Mn, Mx, CNT = _sc_minmax(B_pad, src, dst2)
    CNT2 = CNT.reshape(NPAD, 16)
    return _post(x, A_pad[:n], S1[:n], S2[:n], Mn[:n], Mx[:n], CNT2[:n],
                 W_post[:F], W_post[F:5 * F], W_post[5 * F:9 * F],
                 W_post[9 * F:13 * F], b_post.reshape(1, F),
                 W_lin, b_lin.reshape(1, F))
